# baseline (speedup 1.0000x reference)
import functools

import jax
import jax.numpy as jnp
from jax import lax
from jax.experimental import pallas as pl
from jax.experimental.pallas import tpu as pltpu
from jax.experimental.pallas import tpu_sc as plsc

TOK_BLK = 256
NCHUNK = 4


def _bf16_rne(v):
    u = lax.bitcast_convert_type(v, jnp.uint32)
    lsb = (u >> 16) & jnp.uint32(1)
    u = (u + (jnp.uint32(0x7FFF) + lsb)) & jnp.uint32(0xFFFF0000)
    return lax.bitcast_convert_type(u, jnp.float32)


def _nn_kernel(z_ref, w_ref, xsq_ref, idx_ref):
    zb = z_ref[...]
    wb = w_ref[...]
    x_sq = xsq_ref[...]
    w_sq = jnp.sum(wb * wb, axis=1)[None, :]
    dot = lax.dot_general(zb, wb, (((1,), (1,)), ((), ())),
                          preferred_element_type=jnp.float32)
    d2 = x_sq - 2.0 * dot + w_sq
    dist = jnp.sqrt(jnp.maximum(d2, 0.0))
    k_total = dist.shape[1]
    chunk = k_total // NCHUNK
    accv = acci = None
    for c in range(NCHUNK):
        dc = dist[:, c * chunk:(c + 1) * chunk]
        m = jnp.min(dc, axis=1)
        iota = lax.broadcasted_iota(jnp.int32, dc.shape, 1) + jnp.int32(c * chunk)
        i = jnp.min(jnp.where(dc == m[:, None], iota, jnp.int32(k_total)), axis=1)
        if accv is None:
            accv, acci = _bf16_rne(m), i
        else:
            new_wins = (m < accv) | ((m == accv) & (i < acci))
            accv = _bf16_rne(jnp.where(new_wins, m, accv))
            acci = jnp.where(new_wins, i, acci)
    idx_ref[0, 0, :] = acci


def _nearest_codes(flat, codebook, x_sq):
    n, c = flat.shape
    grid = n // TOK_BLK
    out = pl.pallas_call(
        _nn_kernel,
        grid=(grid,),
        in_specs=[
            pl.BlockSpec((TOK_BLK, c), lambda i: (i, 0)),
            pl.BlockSpec(codebook.shape, lambda i: (0, 0)),
            pl.BlockSpec((TOK_BLK, 1), lambda i: (i, 0)),
        ],
        out_specs=pl.BlockSpec((1, 1, TOK_BLK), lambda i: (i, 0, 0)),
        out_shape=jax.ShapeDtypeStruct((grid, 1, TOK_BLK), jnp.int32),
    )(flat, codebook, x_sq)
    return out.reshape(n)


def _sc_gather(table, idx):
    info = plsc.get_sparse_core_info()
    nw = info.num_cores * info.num_subcores
    b, d = idx.shape[0], table.shape[1]
    b_per_w = b // nw
    mesh = plsc.VectorSubcoreMesh(core_axis_name="c", subcore_axis_name="s")

    @functools.partial(
        pl.kernel, mesh=mesh,
        out_type=jax.ShapeDtypeStruct((b, d), jnp.float32),
        compiler_params=pltpu.CompilerParams(use_tc_tiling_on_sc=False),
        scratch_types=[
            pltpu.VMEM((b_per_w,), jnp.int32),
            pltpu.VMEM((b_per_w, d), jnp.float32),
            pltpu.SemaphoreType.DMA,
        ],
    )
    def gather_k(table_hbm, idx_hbm, out_hbm, idx_v, rows_v, sem):
        wid = lax.axis_index("s") * info.num_cores + lax.axis_index("c")
        base = wid * b_per_w
        pltpu.sync_copy(idx_hbm.at[pl.ds(base, b_per_w)], idx_v)
        pltpu.async_copy(table_hbm.at[idx_v], rows_v, sem).wait()
        pltpu.sync_copy(rows_v, out_hbm.at[pl.ds(base, b_per_w)])

    return gather_k(table, idx)


def kernel(z_e, W):
    b, c, h, w = z_e.shape
    z = jnp.transpose(z_e, (0, 2, 3, 1))
    flat = z.reshape(-1, c)
    x_sq = jnp.sum(flat * flat, axis=1, keepdims=True)
    idx = _nearest_codes(flat, W, x_sq)
    zq_flat = _sc_gather(W, idx)
    z_q = zq_flat.reshape(b, h, w, c)
    z_q_st = z + (z_q - z)
    z_q_out = jnp.transpose(z_q, (0, 3, 1, 2))
    z_q_st_out = jnp.transpose(z_q_st, (0, 3, 1, 2))
    return (z_q_st_out, z_q_out, idx.reshape(b, h * w))


# speedup vs baseline: 1.0508x; 1.0508x over previous
import functools

import jax
import jax.numpy as jnp
from jax import lax
from jax.experimental import pallas as pl
from jax.experimental.pallas import tpu as pltpu
from jax.experimental.pallas import tpu_sc as plsc

TOK_BLK = 256
NCHUNK = 4


def _bf16_rne(v):
    u = lax.bitcast_convert_type(v, jnp.uint32)
    lsb = (u >> 16) & jnp.uint32(1)
    u = (u + (jnp.uint32(0x7FFF) + lsb)) & jnp.uint32(0xFFFF0000)
    return lax.bitcast_convert_type(u, jnp.float32)


def _nn_kernel(z_ref, w_ref, xsq_ref, wsq_ref, idx_ref):
    zb = z_ref[...]
    wb = w_ref[...]
    x_sq = xsq_ref[...]
    w_sq = wsq_ref[...]
    dot = lax.dot_general(zb, wb, (((1,), (1,)), ((), ())),
                          preferred_element_type=jnp.float32)
    d2 = x_sq - 2.0 * dot + w_sq
    dist = jnp.sqrt(jnp.maximum(d2, 0.0))
    k_total = dist.shape[1]
    chunk = k_total // NCHUNK
    accv = acci = None
    for c in range(NCHUNK):
        dc = dist[:, c * chunk:(c + 1) * chunk]
        m = jnp.min(dc, axis=1)
        iota = lax.broadcasted_iota(jnp.int32, dc.shape, 1) + jnp.int32(c * chunk)
        i = jnp.min(jnp.where(dc == m[:, None], iota, jnp.int32(k_total)), axis=1)
        if accv is None:
            accv, acci = _bf16_rne(m), i
        else:
            new_wins = (m < accv) | ((m == accv) & (i < acci))
            accv = _bf16_rne(jnp.where(new_wins, m, accv))
            acci = jnp.where(new_wins, i, acci)
    idx_ref[0, 0, :] = acci


def _nearest_codes(flat, codebook, x_sq, w_sq):
    n, c = flat.shape
    grid = n // TOK_BLK
    out = pl.pallas_call(
        _nn_kernel,
        grid=(grid,),
        in_specs=[
            pl.BlockSpec((TOK_BLK, c), lambda i: (i, 0)),
            pl.BlockSpec(codebook.shape, lambda i: (0, 0)),
            pl.BlockSpec((TOK_BLK, 1), lambda i: (i, 0)),
            pl.BlockSpec(w_sq.shape, lambda i: (0, 0)),
        ],
        out_specs=pl.BlockSpec((1, 1, TOK_BLK), lambda i: (i, 0, 0)),
        out_shape=jax.ShapeDtypeStruct((grid, 1, TOK_BLK), jnp.int32),
        compiler_params=pltpu.CompilerParams(
            dimension_semantics=("parallel",)),
    )(flat, codebook, x_sq, w_sq)
    return out.reshape(n)


def _sc_gather(table, idx):
    info = plsc.get_sparse_core_info()
    nw = info.num_cores * info.num_subcores
    b, d = idx.shape[0], table.shape[1]
    b_per_w = b // nw
    mesh = plsc.VectorSubcoreMesh(core_axis_name="c", subcore_axis_name="s")

    g = 128
    @functools.partial(
        pl.kernel, mesh=mesh,
        out_type=jax.ShapeDtypeStruct((b, d), jnp.float32),
        compiler_params=pltpu.CompilerParams(use_tc_tiling_on_sc=False),
        scratch_types=[
            pltpu.VMEM((g,), jnp.int32),
            pltpu.VMEM((g, d), jnp.float32),
            pltpu.SemaphoreType.DMA,
        ],
    )
    def gather_k(table_hbm, idx_hbm, out_hbm, idx_v, rows_v, sem):
        wid = lax.axis_index("s") * info.num_cores + lax.axis_index("c")
        base = wid * b_per_w
        for h in range(b_per_w // g):
            off = base + h * g
            pltpu.sync_copy(idx_hbm.at[pl.ds(off, g)], idx_v)
            pltpu.async_copy(table_hbm.at[idx_v], rows_v, sem).wait()
            pltpu.sync_copy(rows_v, out_hbm.at[pl.ds(off, g)])

    return gather_k(table, idx)


def kernel(z_e, W):
    b, c, h, w = z_e.shape
    z = jnp.transpose(z_e, (0, 2, 3, 1))
    flat = z.reshape(-1, c)
    x_sq = jnp.sum(flat * flat, axis=1, keepdims=True)
    w_sq = jnp.sum(W * W, axis=1)[None, :]
    idx = _nearest_codes(flat, W, x_sq, w_sq)
    zq_flat = _sc_gather(W, idx)
    z_q = zq_flat.reshape(b, h, w, c)
    z_q_st = z + (z_q - z)
    z_q_out = jnp.transpose(z_q, (0, 3, 1, 2))
    z_q_st_out = jnp.transpose(z_q_st, (0, 3, 1, 2))
    return (z_q_st_out, z_q_out, idx.reshape(b, h * w))


# speedup vs baseline: 1.1024x; 1.0491x over previous
import functools

import jax
import jax.numpy as jnp
from jax import lax
from jax.experimental import pallas as pl
from jax.experimental.pallas import tpu as pltpu
from jax.experimental.pallas import tpu_sc as plsc

TOK_BLK = 512
NCHUNK = 4


def _bf16_rne(v):
    u = lax.bitcast_convert_type(v, jnp.uint32)
    lsb = (u >> 16) & jnp.uint32(1)
    u = (u + (jnp.uint32(0x7FFF) + lsb)) & jnp.uint32(0xFFFF0000)
    return lax.bitcast_convert_type(u, jnp.float32)


def _nn_kernel(z_ref, w_ref, xsq_ref, wsq_ref, idx_ref):
    zb = z_ref[...]
    wb = w_ref[...]
    x_sq = xsq_ref[...]
    w_sq = wsq_ref[...]
    dot = lax.dot_general(zb, wb, (((1,), (1,)), ((), ())),
                          preferred_element_type=jnp.float32)
    d2 = x_sq - 2.0 * dot + w_sq
    dist = jnp.sqrt(jnp.maximum(d2, 0.0))
    k_total = dist.shape[1]
    chunk = k_total // NCHUNK
    accv = acci = None
    for c in range(NCHUNK):
        dc = dist[:, c * chunk:(c + 1) * chunk]
        m = jnp.min(dc, axis=1)
        iota = lax.broadcasted_iota(jnp.int32, dc.shape, 1) + jnp.int32(c * chunk)
        i = jnp.min(jnp.where(dc == m[:, None], iota, jnp.int32(k_total)), axis=1)
        if accv is None:
            accv, acci = _bf16_rne(m), i
        else:
            new_wins = (m < accv) | ((m == accv) & (i < acci))
            accv = _bf16_rne(jnp.where(new_wins, m, accv))
            acci = jnp.where(new_wins, i, acci)
    idx_ref[0, 0, :] = acci


def _nearest_codes(flat, codebook, x_sq, w_sq):
    n, c = flat.shape
    grid = n // TOK_BLK
    out = pl.pallas_call(
        _nn_kernel,
        grid=(grid,),
        in_specs=[
            pl.BlockSpec((TOK_BLK, c), lambda i: (i, 0)),
            pl.BlockSpec(codebook.shape, lambda i: (0, 0)),
            pl.BlockSpec((TOK_BLK, 1), lambda i: (i, 0)),
            pl.BlockSpec(w_sq.shape, lambda i: (0, 0)),
        ],
        out_specs=pl.BlockSpec((1, 1, TOK_BLK), lambda i: (i, 0, 0)),
        out_shape=jax.ShapeDtypeStruct((grid, 1, TOK_BLK), jnp.int32),
        compiler_params=pltpu.CompilerParams(
            dimension_semantics=("parallel",)),
    )(flat, codebook, x_sq, w_sq)
    return out.reshape(n)


def _sc_gather(table, idx):
    info = plsc.get_sparse_core_info()
    nw = info.num_cores * info.num_subcores
    b, d = idx.shape[0], table.shape[1]
    b_per_w = b // nw
    mesh = plsc.VectorSubcoreMesh(core_axis_name="c", subcore_axis_name="s")

    g = 128
    @functools.partial(
        pl.kernel, mesh=mesh,
        out_type=jax.ShapeDtypeStruct((b, d), jnp.float32),
        compiler_params=pltpu.CompilerParams(use_tc_tiling_on_sc=False),
        scratch_types=[
            pltpu.VMEM((g,), jnp.int32),
            pltpu.VMEM((g, d), jnp.float32),
            pltpu.SemaphoreType.DMA,
        ],
    )
    def gather_k(table_hbm, idx_hbm, out_hbm, idx_v, rows_v, sem):
        wid = lax.axis_index("s") * info.num_cores + lax.axis_index("c")
        base = wid * b_per_w
        for h in range(b_per_w // g):
            off = base + h * g
            pltpu.sync_copy(idx_hbm.at[pl.ds(off, g)], idx_v)
            pltpu.async_copy(table_hbm.at[idx_v], rows_v, sem).wait()
            pltpu.sync_copy(rows_v, out_hbm.at[pl.ds(off, g)])

    return gather_k(table, idx)


def kernel(z_e, W):
    b, c, h, w = z_e.shape
    z = jnp.transpose(z_e, (0, 2, 3, 1))
    flat = z.reshape(-1, c)
    x_sq = jnp.sum(flat * flat, axis=1, keepdims=True)
    w_sq = jnp.sum(W * W, axis=1)[None, :]
    idx = _nearest_codes(flat, W, x_sq, w_sq)
    zq_flat = _sc_gather(W, idx)
    z_q = zq_flat.reshape(b, h, w, c)
    z_q_st = z + (z_q - z)
    z_q_out = jnp.transpose(z_q, (0, 3, 1, 2))
    z_q_st_out = jnp.transpose(z_q_st, (0, 3, 1, 2))
    return (z_q_st_out, z_q_out, idx.reshape(b, h * w))


# speedup vs baseline: 1.1299x; 1.0250x over previous
import functools

import jax
import jax.numpy as jnp
from jax import lax
from jax.experimental import pallas as pl
from jax.experimental.pallas import tpu as pltpu
from jax.experimental.pallas import tpu_sc as plsc

TOK_BLK = 1024
NCHUNK = 4


def _bf16_rne(v):
    u = lax.bitcast_convert_type(v, jnp.uint32)
    lsb = (u >> 16) & jnp.uint32(1)
    u = (u + (jnp.uint32(0x7FFF) + lsb)) & jnp.uint32(0xFFFF0000)
    return lax.bitcast_convert_type(u, jnp.float32)


def _nn_kernel(z_ref, w_ref, xsq_ref, wsq_ref, idx_ref):
    zb = z_ref[...]
    wb = w_ref[...]
    x_sq = xsq_ref[...]
    w_sq = wsq_ref[...]
    dot = lax.dot_general(zb, wb, (((1,), (1,)), ((), ())),
                          preferred_element_type=jnp.float32)
    d2 = x_sq - 2.0 * dot + w_sq
    dist = jnp.sqrt(jnp.maximum(d2, 0.0))
    k_total = dist.shape[1]
    chunk = k_total // NCHUNK
    accv = acci = None
    for c in range(NCHUNK):
        dc = dist[:, c * chunk:(c + 1) * chunk]
        m = jnp.min(dc, axis=1)
        iota = lax.broadcasted_iota(jnp.int32, dc.shape, 1) + jnp.int32(c * chunk)
        i = jnp.min(jnp.where(dc == m[:, None], iota, jnp.int32(k_total)), axis=1)
        if accv is None:
            accv, acci = _bf16_rne(m), i
        else:
            new_wins = (m < accv) | ((m == accv) & (i < acci))
            accv = _bf16_rne(jnp.where(new_wins, m, accv))
            acci = jnp.where(new_wins, i, acci)
    idx_ref[0, 0, :] = acci


def _nearest_codes(flat, codebook, x_sq, w_sq):
    n, c = flat.shape
    grid = n // TOK_BLK
    out = pl.pallas_call(
        _nn_kernel,
        grid=(grid,),
        in_specs=[
            pl.BlockSpec((TOK_BLK, c), lambda i: (i, 0)),
            pl.BlockSpec(codebook.shape, lambda i: (0, 0)),
            pl.BlockSpec((TOK_BLK, 1), lambda i: (i, 0)),
            pl.BlockSpec(w_sq.shape, lambda i: (0, 0)),
        ],
        out_specs=pl.BlockSpec((1, 1, TOK_BLK), lambda i: (i, 0, 0)),
        out_shape=jax.ShapeDtypeStruct((grid, 1, TOK_BLK), jnp.int32),
        compiler_params=pltpu.CompilerParams(
            dimension_semantics=("parallel",)),
    )(flat, codebook, x_sq, w_sq)
    return out.reshape(n)


def _sc_gather(table, idx):
    info = plsc.get_sparse_core_info()
    nw = info.num_cores * info.num_subcores
    b, d = idx.shape[0], table.shape[1]
    b_per_w = b // nw
    mesh = plsc.VectorSubcoreMesh(core_axis_name="c", subcore_axis_name="s")

    g = 128
    @functools.partial(
        pl.kernel, mesh=mesh,
        out_type=jax.ShapeDtypeStruct((b, d), jnp.float32),
        compiler_params=pltpu.CompilerParams(use_tc_tiling_on_sc=False),
        scratch_types=[
            pltpu.VMEM((g,), jnp.int32),
            pltpu.VMEM((g, d), jnp.float32),
            pltpu.SemaphoreType.DMA,
        ],
    )
    def gather_k(table_hbm, idx_hbm, out_hbm, idx_v, rows_v, sem):
        wid = lax.axis_index("s") * info.num_cores + lax.axis_index("c")
        base = wid * b_per_w
        for h in range(b_per_w // g):
            off = base + h * g
            pltpu.sync_copy(idx_hbm.at[pl.ds(off, g)], idx_v)
            pltpu.async_copy(table_hbm.at[idx_v], rows_v, sem).wait()
            pltpu.sync_copy(rows_v, out_hbm.at[pl.ds(off, g)])

    return gather_k(table, idx)


def kernel(z_e, W):
    b, c, h, w = z_e.shape
    z = jnp.transpose(z_e, (0, 2, 3, 1))
    flat = z.reshape(-1, c)
    x_sq = jnp.sum(flat * flat, axis=1, keepdims=True)
    w_sq = jnp.sum(W * W, axis=1)[None, :]
    idx = _nearest_codes(flat, W, x_sq, w_sq)
    zq_flat = _sc_gather(W, idx)
    z_q = zq_flat.reshape(b, h, w, c)
    z_q_st = z + (z_q - z)
    z_q_out = jnp.transpose(z_q, (0, 3, 1, 2))
    z_q_st_out = jnp.transpose(z_q_st, (0, 3, 1, 2))
    return (z_q_st_out, z_q_out, idx.reshape(b, h * w))
